# baseline (speedup 1.0000x reference)
import functools

import jax
import jax.numpy as jnp
from jax import lax
from jax.experimental import pallas as pl
from jax.experimental.pallas import tpu as pltpu
from jax.experimental.pallas import tpu_sc as plsc

VOCAB = 1000001
EMBED = 64
BATCH = 4096
SEQ = 50
NUM_CORES = 2
NUM_SUBCORES = 16
NW = NUM_CORES * NUM_SUBCORES

FULL_TCOLS = 7812
TAIL_START = FULL_TCOLS * 128
N_TAIL = VOCAB - TAIL_START
PACKED_ROWS = 500008
BLOCKS_EACH = 244


def _iota16():
    return lax.iota(jnp.int32, 16)


@jax.jit
def _pack_table(tbl_t):
    mesh = plsc.VectorSubcoreMesh(core_axis_name="c", subcore_axis_name="s")

    @functools.partial(
        pl.kernel,
        mesh=mesh,
        out_type=jax.ShapeDtypeStruct((PACKED_ROWS, 128), jnp.float32),
        scratch_types=[
            pltpu.VMEM((2, EMBED, 128), jnp.float32),
            pltpu.VMEM((2, 64, 128), jnp.float32),
            pltpu.SemaphoreType.DMA((2,)),
            pltpu.SemaphoreType.DMA((2,)),
        ],
        compiler_params=pltpu.CompilerParams(use_tc_tiling_on_sc=True, needs_layout_passes=False),
    )
    def pack_kernel(tbl_hbm, out_hbm, src_v, dst_v, rsem, wsem):
        wid = lax.axis_index("s") * NUM_CORES + lax.axis_index("c")

        def rcopy(jc, p):
            return pltpu.make_async_copy(
                tbl_hbm.at[:, pl.ds(jc * 128, 128)], src_v.at[p], rsem.at[p])

        def wcopy(jc, p):
            return pltpu.make_async_copy(
                dst_v.at[p], out_hbm.at[pl.ds(jc * 64, 64)], wsem.at[p])

        def transpose_block(p):
            for g in range(8):
                d_idx = 16 * (g % 4) + _iota16()
                for kk in range(64):
                    c_idx = jnp.full((16,), 2 * kk + g // 4, jnp.int32)
                    vals = plsc.load_gather(src_v.at[p], [d_idx, c_idx])
                    dst_v[p, kk, pl.ds(16 * g, 16)] = vals

        rcopy(wid, 0).start()

        def body(t, carry):
            p = lax.rem(t, 2)
            jc = wid + NW * t
            rcopy(jc, p).wait()

            @pl.when(t + 1 < BLOCKS_EACH)
            def _():
                rcopy(jc + NW, 1 - p).start()

            @pl.when(t >= 2)
            def _():
                wcopy(jc - 2 * NW, p).wait()

            transpose_block(p)
            wcopy(jc, p).start()
            return carry

        lax.fori_loop(0, BLOCKS_EACH, body, 0)
        wcopy(wid + NW * (BLOCKS_EACH - 2), 0).wait()
        wcopy(wid + NW * (BLOCKS_EACH - 1), 1).wait()

        @pl.when(wid < 4)
        def _():
            jc = FULL_TCOLS - 4 + wid
            pltpu.sync_copy(tbl_hbm.at[:, pl.ds(jc * 128, 128)], src_v.at[0])
            transpose_block(0)
            pltpu.sync_copy(dst_v.at[0], out_hbm.at[pl.ds(jc * 64, 64)])

    return pack_kernel(tbl_t)


@jax.jit
def _lookup(ids_t, packed, tail):
    mesh = plsc.VectorSubcoreMesh(core_axis_name="c", subcore_axis_name="s")

    @functools.partial(
        pl.kernel,
        mesh=mesh,
        out_type=jax.ShapeDtypeStruct((SEQ, EMBED, BATCH), jnp.float32),
        scratch_types=[
            pltpu.VMEM((SEQ, 128), jnp.int32),
            pltpu.VMEM((SEQ, 128), jnp.int32),
            pltpu.VMEM((SEQ, 128), jnp.int32),
            pltpu.VMEM((N_TAIL, EMBED), jnp.float32),
            pltpu.VMEM((3, 128, 128), jnp.float32),
            pltpu.VMEM((2, EMBED, 128), jnp.float32),
            pltpu.SemaphoreType.DMA((3,)),
            pltpu.SemaphoreType.DMA((2,)),
        ],
        compiler_params=pltpu.CompilerParams(use_tc_tiling_on_sc=True, needs_layout_passes=False),
    )
    def lookup_kernel(ids_hbm, tbl_hbm, tail_hbm, out_hbm, v_v, k_v, h_v,
                      tail_v, rows_v, o_v, gsem, wsem):
        wid = lax.axis_index("s") * NUM_CORES + lax.axis_index("c")
        b0 = wid * (BATCH // NW)

        pltpu.sync_copy(ids_hbm.at[:, pl.ds(b0, 128)], v_v)
        pltpu.sync_copy(tail_hbm, tail_v)
        for s in range(SEQ):
            for g in range(8):
                v = v_v[s, pl.ds(16 * g, 16)]
                k_v[s, pl.ds(16 * g, 16)] = lax.shift_right_logical(v, 1)
                h_v[s, pl.ds(16 * g, 16)] = lax.mul(
                    lax.rem(v, 2), jnp.full((16,), 64, jnp.int32))

        def gcopy(s, p):
            return pltpu.make_async_copy(
                tbl_hbm.at[k_v.at[s]], rows_v.at[p], gsem.at[p])

        def wcopy(s, q):
            return pltpu.make_async_copy(
                o_v.at[q], out_hbm.at[s, :, pl.ds(b0, 128)], wsem.at[q])

        gcopy(0, 0).start()
        gcopy(1, 1).start()

        def body(s, carry):
            p = lax.rem(s, 3)
            q = lax.rem(s, 2)
            gcopy(s, p).wait()

            @pl.when(s + 2 < SEQ)
            def _():
                gcopy(s + 2, lax.rem(s + 2, 3)).start()

            @pl.when(s >= 2)
            def _():
                wcopy(s - 2, q).wait()

            for g in range(8):
                i_idx = 16 * g + _iota16()
                hvec = h_v[s, pl.ds(16 * g, 16)]
                for d in range(EMBED):
                    c_idx = hvec + jnp.full((16,), d, jnp.int32)
                    vals = plsc.load_gather(rows_v.at[p], [i_idx, c_idx])
                    o_v[q, d, pl.ds(16 * g, 16)] = vals

            thresh = jnp.full((16,), TAIL_START, jnp.int32)
            n_big = jnp.zeros((), jnp.int32)
            for g in range(8):
                v = v_v[s, pl.ds(16 * g, 16)]
                n_big = n_big + plsc.all_reduce_population_count(
                    v >= thresh)[0]

            @pl.when(n_big > 0)
            def _():
                for g in range(8):
                    v = v_v[s, pl.ds(16 * g, 16)]
                    big = v >= thresh
                    t_idx = lax.max(v - thresh, jnp.zeros((16,), jnp.int32))
                    for d in range(EMBED):
                        cur = o_v[q, d, pl.ds(16 * g, 16)]
                        fix = plsc.load_gather(
                            tail_v, [t_idx, jnp.full((16,), d, jnp.int32)])
                        o_v[q, d, pl.ds(16 * g, 16)] = jnp.where(
                            big, fix, cur)

            wcopy(s, q).start()
            return carry

        lax.fori_loop(0, SEQ, body, 0)
        wcopy(SEQ - 2, 0).wait()
        wcopy(SEQ - 1, 1).wait()

    return lookup_kernel(ids_t, packed, tail)


def kernel(input_ids, table):
    ids_t = input_ids.astype(jnp.int32).T
    packed = _pack_table(table.T)
    tail = table[TAIL_START:]
    out_t = _lookup(ids_t, packed, tail)
    return jnp.transpose(out_t, (2, 0, 1))


# speedup vs baseline: 4.9833x; 4.9833x over previous
import functools

import jax
import jax.numpy as jnp
from jax import lax
from jax.experimental import pallas as pl
from jax.experimental.pallas import tpu as pltpu
from jax.experimental.pallas import tpu_sc as plsc

VOCAB = 1000001
EMBED = 64
BATCH = 4096
SEQ = 50
NUM_CORES = 2
NUM_SUBCORES = 16
NW = NUM_CORES * NUM_SUBCORES

FULL_TCOLS = 7812
TAIL_START = FULL_TCOLS * 128
N_TAIL = VOCAB - TAIL_START
PACKED_ROWS = 500008
BLOCKS_EACH = 244


def _iota16():
    return lax.iota(jnp.int32, 16)


@jax.jit
def _pack_table(tbl_t):
    mesh = plsc.VectorSubcoreMesh(core_axis_name="c", subcore_axis_name="s")

    @functools.partial(
        pl.kernel,
        mesh=mesh,
        out_type=jax.ShapeDtypeStruct((PACKED_ROWS, 128), jnp.float32),
        scratch_types=[
            pltpu.VMEM((2, EMBED, 128), jnp.float32),
            pltpu.VMEM((2, 64, 128), jnp.float32),
            pltpu.SemaphoreType.DMA((2,)),
            pltpu.SemaphoreType.DMA((2,)),
        ],
        compiler_params=pltpu.CompilerParams(use_tc_tiling_on_sc=True, needs_layout_passes=False),
    )
    def pack_kernel(tbl_hbm, out_hbm, src_v, dst_v, rsem, wsem):
        wid = lax.axis_index("s") * NUM_CORES + lax.axis_index("c")

        def rcopy(jc, p):
            return pltpu.make_async_copy(
                tbl_hbm.at[:, pl.ds(jc * 128, 128)], src_v.at[p], rsem.at[p])

        def wcopy(jc, p):
            return pltpu.make_async_copy(
                dst_v.at[p], out_hbm.at[pl.ds(jc * 64, 64)], wsem.at[p])

        ones16 = jnp.ones((16,), jnp.int32)

        def transpose_block(p):
            for g in range(8):
                d_idx = 16 * (g % 4) + _iota16()
                cbase = g // 4

                @functools.partial(plsc.parallel_loop, 0, 64, unroll=8)
                def _(kk):
                    c_idx = (2 * kk + cbase) * ones16
                    vals = plsc.load_gather(src_v.at[p], [d_idx, c_idx])
                    dst_v[p, kk, pl.ds(16 * g, 16)] = vals

        rcopy(wid, 0).start()

        def body(t, carry):
            p = lax.rem(t, 2)
            jc = wid + NW * t
            rcopy(jc, p).wait()

            @pl.when(t + 1 < BLOCKS_EACH)
            def _():
                rcopy(jc + NW, 1 - p).start()

            @pl.when(t >= 2)
            def _():
                wcopy(jc - 2 * NW, p).wait()

            transpose_block(p)
            wcopy(jc, p).start()
            return carry

        lax.fori_loop(0, BLOCKS_EACH, body, 0)
        wcopy(wid + NW * (BLOCKS_EACH - 2), 0).wait()
        wcopy(wid + NW * (BLOCKS_EACH - 1), 1).wait()

        @pl.when(wid < 4)
        def _():
            jc = FULL_TCOLS - 4 + wid
            pltpu.sync_copy(tbl_hbm.at[:, pl.ds(jc * 128, 128)], src_v.at[0])
            transpose_block(0)
            pltpu.sync_copy(dst_v.at[0], out_hbm.at[pl.ds(jc * 64, 64)])

    return pack_kernel(tbl_t)


@jax.jit
def _lookup(ids_t, packed, tail):
    mesh = plsc.VectorSubcoreMesh(core_axis_name="c", subcore_axis_name="s")

    @functools.partial(
        pl.kernel,
        mesh=mesh,
        out_type=jax.ShapeDtypeStruct((SEQ, EMBED, BATCH), jnp.float32),
        scratch_types=[
            pltpu.VMEM((SEQ, 128), jnp.int32),
            pltpu.VMEM((SEQ, 128), jnp.int32),
            pltpu.VMEM((SEQ, 128), jnp.int32),
            pltpu.VMEM((N_TAIL, EMBED), jnp.float32),
            pltpu.VMEM((3, 128, 128), jnp.float32),
            pltpu.VMEM((2, EMBED, 128), jnp.float32),
            pltpu.SemaphoreType.DMA((3,)),
            pltpu.SemaphoreType.DMA((2,)),
        ],
        compiler_params=pltpu.CompilerParams(use_tc_tiling_on_sc=True, needs_layout_passes=False),
    )
    def lookup_kernel(ids_hbm, tbl_hbm, tail_hbm, out_hbm, v_v, k_v, h_v,
                      tail_v, rows_v, o_v, gsem, wsem):
        wid = lax.axis_index("s") * NUM_CORES + lax.axis_index("c")
        b0 = wid * (BATCH // NW)

        pltpu.sync_copy(ids_hbm.at[:, pl.ds(b0, 128)], v_v)
        pltpu.sync_copy(tail_hbm, tail_v)
        for s in range(SEQ):
            for g in range(8):
                v = v_v[s, pl.ds(16 * g, 16)]
                k_v[s, pl.ds(16 * g, 16)] = lax.shift_right_logical(v, 1)
                h_v[s, pl.ds(16 * g, 16)] = lax.mul(
                    lax.rem(v, 2), jnp.full((16,), 64, jnp.int32))

        def gcopy(s, p):
            return pltpu.make_async_copy(
                tbl_hbm.at[k_v.at[s]], rows_v.at[p], gsem.at[p])

        def wcopy(s, q):
            return pltpu.make_async_copy(
                o_v.at[q], out_hbm.at[s, :, pl.ds(b0, 128)], wsem.at[q])

        gcopy(0, 0).start()
        gcopy(1, 1).start()

        def body(s, carry):
            p = lax.rem(s, 3)
            q = lax.rem(s, 2)
            gcopy(s, p).wait()

            @pl.when(s + 2 < SEQ)
            def _():
                gcopy(s + 2, lax.rem(s + 2, 3)).start()

            @pl.when(s >= 2)
            def _():
                wcopy(s - 2, q).wait()

            for g in range(8):
                i_idx = 16 * g + _iota16()
                hvec = h_v[s, pl.ds(16 * g, 16)]

                @functools.partial(plsc.parallel_loop, 0, EMBED, unroll=8)
                def _(d):
                    c_idx = hvec + d
                    vals = plsc.load_gather(rows_v.at[p], [i_idx, c_idx])
                    o_v[q, d, pl.ds(16 * g, 16)] = vals

            thresh = jnp.full((16,), TAIL_START, jnp.int32)
            n_big = jnp.zeros((), jnp.int32)
            for g in range(8):
                v = v_v[s, pl.ds(16 * g, 16)]
                n_big = n_big + plsc.all_reduce_population_count(
                    v >= thresh)[0]

            @pl.when(n_big > 0)
            def _():
                for g in range(8):
                    v = v_v[s, pl.ds(16 * g, 16)]
                    big = v >= thresh
                    t_idx = lax.max(v - thresh, jnp.zeros((16,), jnp.int32))
                    for d in range(EMBED):
                        cur = o_v[q, d, pl.ds(16 * g, 16)]
                        fix = plsc.load_gather(
                            tail_v, [t_idx, jnp.full((16,), d, jnp.int32)])
                        o_v[q, d, pl.ds(16 * g, 16)] = jnp.where(
                            big, fix, cur)

            wcopy(s, q).start()
            return carry

        lax.fori_loop(0, SEQ, body, 0)
        wcopy(SEQ - 2, 0).wait()
        wcopy(SEQ - 1, 1).wait()

    return lookup_kernel(ids_t, packed, tail)


def kernel(input_ids, table):
    ids_t = input_ids.astype(jnp.int32).T
    packed = _pack_table(table.T)
    tail = table[TAIL_START:]
    out_t = _lookup(ids_t, packed, tail)
    return jnp.transpose(out_t, (2, 0, 1))
